# baseline (speedup 1.0000x reference)
import functools

import jax
import jax.numpy as jnp
from jax import lax
from jax.experimental import pallas as pl
from jax.experimental.pallas import tpu as pltpu
from jax.experimental.pallas import tpu_sc as plsc

T = 32768
D = 768
H = 768
E = 64
G4 = 4 * E

TB = 2048

NW = 32
RW = T // NW
CH = 16
NCH = RW // CH


def _logits_body(mm_ref, qf_ref, wqe1_ref, bqe1_ref, wqe2_ref, bqe2_ref,
                 wfg_ref, bfg_ref, wg1_ref, wg2_ref, logits_ref):
    f32 = jnp.float32
    q = jnp.dot(qf_ref[...], wqe1_ref[...], preferred_element_type=f32)
    q = jnp.maximum(q + bqe1_ref[...], 0.0)
    q = jnp.dot(q, wqe2_ref[...], preferred_element_type=f32) + bqe2_ref[...]

    h = jnp.dot(mm_ref[...], wfg_ref[0:H, :], preferred_element_type=f32)
    h = h + jnp.dot(q, wfg_ref[H:2 * H, :], preferred_element_type=f32)
    h = jnp.maximum(h + bfg_ref[...], 0.0)

    g = jnp.tanh(jnp.dot(h, wg1_ref[...], preferred_element_type=f32))
    logits_ref[...] = jnp.dot(g, wg2_ref[...], preferred_element_type=f32)


_GATHER_DNUMS = lax.GatherDimensionNumbers(
    offset_dims=(), collapsed_slice_dims=(0,), start_index_map=(0,))


def _perm(v, idx):
    return lax.gather(v, idx.reshape(16, 1), _GATHER_DNUMS, (1,),
                      mode=lax.GatherScatterMode.PROMISE_IN_BOUNDS)


def _bfly(v, op, iota):
    for s in (1, 2, 4, 8):
        v = op(v, _perm(v, iota ^ s))
    return v


def _tail_sc_body(lg_hbm, ew_hbm, tk_hbm, lg_v, ew_v, tk_v):
    wid = lax.axis_index("s") * 2 + lax.axis_index("c")
    base = wid * RW
    iota = lax.iota(jnp.int32, 16)
    lane = [iota == k for k in range(4)]

    def chunk(ci, carry):
        row0 = base + ci * CH
        pltpu.sync_copy(lg_hbm.at[pl.ds(row0, CH)], lg_v)
        for r in range(CH):
            x = [lg_v[r, pl.ds(k * 16, 16)] for k in range(4)]
            idx = [iota + (16 * k) for k in range(4)]
            rdx = [(E - 16 * k) - iota for k in range(4)]
            m = jnp.maximum(jnp.maximum(x[0], x[1]), jnp.maximum(x[2], x[3]))
            m1 = _bfly(m, jnp.maximum, iota)
            ex = [jnp.exp(xk - m1) for xk in x]
            z = _bfly(ex[0] + ex[1] + ex[2] + ex[3], jnp.add, iota)
            inv = 1.0 / z
            for k in range(4):
                ew_v[r, pl.ds(k * 16, 16)] = ex[k] * inv
            cand = [jnp.where(x[k] == m1, rdx[k], 0) for k in range(4)]
            i1 = E - _bfly(jnp.maximum(jnp.maximum(cand[0], cand[1]),
                                   jnp.maximum(cand[2], cand[3])), jnp.maximum, iota)
            exm = [jnp.where(idx[k] == i1, -1.0, ex[k]) for k in range(4)]
            t2 = _bfly(jnp.maximum(jnp.maximum(exm[0], exm[1]),
                                  jnp.maximum(exm[2], exm[3])), jnp.maximum, iota)
            cand2 = [jnp.where(exm[k] == t2, rdx[k], 0) for k in range(4)]
            i2 = E - _bfly(jnp.maximum(jnp.maximum(cand2[0], cand2[1]),
                                   jnp.maximum(cand2[2], cand2[3])), jnp.maximum, iota)
            denom = 1.0 + t2 + 1e-6 * z
            w1 = 1.0 / denom
            w2 = t2 / denom
            packed = jnp.where(lane[0], w1,
                               jnp.where(lane[1], w2,
                                         jnp.where(lane[2], i1.astype(jnp.float32),
                                                   jnp.where(lane[3], i2.astype(jnp.float32),
                                                             0.0))))
            tk_v[r, :] = packed
        pltpu.sync_copy(ew_v, ew_hbm.at[pl.ds(row0, CH)])
        pltpu.sync_copy(tk_v, tk_hbm.at[pl.ds(row0, CH)])
        return carry

    lax.fori_loop(0, NCH, chunk, 0)


@functools.partial(jax.jit, static_argnames=("interpret",))
def _router(mm, qf, W_qe1, b_qe1, W_qe2, b_qe2, W_fg, b_fg, W_g1, W_g2,
            interpret=False):
    tok = lambda i: (i, 0)
    rep = lambda i: (0, 0)
    logits = pl.pallas_call(
        _logits_body,
        grid=(T // TB,),
        in_specs=[
            pl.BlockSpec((TB, H), tok),
            pl.BlockSpec((TB, D), tok),
            pl.BlockSpec((D, H), rep),
            pl.BlockSpec((1, H), rep),
            pl.BlockSpec((H, H), rep),
            pl.BlockSpec((1, H), rep),
            pl.BlockSpec((2 * H, H), rep),
            pl.BlockSpec((1, H), rep),
            pl.BlockSpec((H, G4), rep),
            pl.BlockSpec((G4, E), rep),
        ],
        out_specs=pl.BlockSpec((TB, E), tok),
        out_shape=jax.ShapeDtypeStruct((T, E), jnp.float32),
        interpret=interpret,
    )(mm, qf, W_qe1, b_qe1, W_qe2, b_qe2, W_fg, b_fg, W_g1, W_g2)

    mesh = plsc.VectorSubcoreMesh(core_axis_name="c", subcore_axis_name="s")
    tail = pl.kernel(
        _tail_sc_body,
        mesh=mesh,
        out_type=[
            jax.ShapeDtypeStruct((T, E), jnp.float32),
            jax.ShapeDtypeStruct((T, 16), jnp.float32),
        ],
        scratch_types=[
            pltpu.VMEM((CH, E), jnp.float32),
            pltpu.VMEM((CH, E), jnp.float32),
            pltpu.VMEM((CH, 16), jnp.float32),
        ],
    )
    ew, tk = tail(logits)
    return logits, ew, tk


def kernel(multimodal_feat, query_feat, W_qe1, b_qe1, W_qe2, b_qe2,
           W_fg, b_fg, W_g1, W_g2):
    logits, ew, tk = _router(
        multimodal_feat, query_feat,
        W_qe1, b_qe1.reshape(1, H),
        W_qe2, b_qe2.reshape(1, H),
        W_fg, b_fg.reshape(1, H),
        W_g1, W_g2)
    return (logits, ew, tk[:, 0:2], tk[:, 2:4].astype(jnp.int32))


# speedup vs baseline: 1.4144x; 1.4144x over previous
import functools

import jax
import jax.numpy as jnp
from jax.experimental import pallas as pl

T = 32768
D = 768
H = 768
E = 64
G4 = 4 * E

TB = 2048


def _router_body(mm_ref, qf_ref, wqe1_ref, bqe1_ref, wqe2_ref, bqe2_ref,
                 wfg_ref, bfg_ref, wg1_ref, wg2_ref,
                 logits_ref, ew_ref, tkw_ref, tki_ref):
    f32 = jnp.float32
    q = jnp.dot(qf_ref[...], wqe1_ref[...], preferred_element_type=f32)
    q = jnp.maximum(q + bqe1_ref[...], 0.0)
    q = jnp.dot(q, wqe2_ref[...], preferred_element_type=f32) + bqe2_ref[...]

    h = jnp.dot(mm_ref[...], wfg_ref[0:H, :], preferred_element_type=f32)
    h = h + jnp.dot(q, wfg_ref[H:2 * H, :], preferred_element_type=f32)
    h = jnp.maximum(h + bfg_ref[...], 0.0)

    g = jnp.tanh(jnp.dot(h, wg1_ref[...], preferred_element_type=f32))
    logits = jnp.dot(g, wg2_ref[...], preferred_element_type=f32)
    logits_ref[...] = logits

    m1 = jnp.max(logits, axis=-1, keepdims=True)
    ex = jnp.exp(logits - m1)
    z = jnp.sum(ex, axis=-1, keepdims=True)
    ew_ref[...] = ex / z

    col = jax.lax.broadcasted_iota(jnp.int32, logits.shape, 1)
    i1 = jnp.min(jnp.where(logits == m1, col, E), axis=-1, keepdims=True)
    masked = jnp.where(col == i1, -jnp.inf, logits)
    m2 = jnp.max(masked, axis=-1, keepdims=True)
    i2 = jnp.min(jnp.where(masked == m2, col, E), axis=-1, keepdims=True)

    t2 = jnp.exp(m2 - m1)
    denom = 1.0 + t2 + 1e-6 * z
    tkw_ref[...] = jnp.concatenate([jnp.ones_like(t2), t2], axis=1) / denom
    tki_ref[...] = jnp.concatenate([i1, i2], axis=1)


@functools.partial(jax.jit, static_argnames=("interpret",))
def _router(mm, qf, W_qe1, b_qe1, W_qe2, b_qe2, W_fg, b_fg, W_g1, W_g2,
            interpret=False):
    tok = lambda i: (i, 0)
    rep = lambda i: (0, 0)
    return pl.pallas_call(
        _router_body,
        grid=(T // TB,),
        in_specs=[
            pl.BlockSpec((TB, H), tok),
            pl.BlockSpec((TB, D), tok),
            pl.BlockSpec((D, H), rep),
            pl.BlockSpec((1, H), rep),
            pl.BlockSpec((H, H), rep),
            pl.BlockSpec((1, H), rep),
            pl.BlockSpec((2 * H, H), rep),
            pl.BlockSpec((1, H), rep),
            pl.BlockSpec((H, G4), rep),
            pl.BlockSpec((G4, E), rep),
        ],
        out_specs=[
            pl.BlockSpec((TB, E), tok),
            pl.BlockSpec((TB, E), tok),
            pl.BlockSpec((TB, 2), tok),
            pl.BlockSpec((TB, 2), tok),
        ],
        out_shape=[
            jax.ShapeDtypeStruct((T, E), jnp.float32),
            jax.ShapeDtypeStruct((T, E), jnp.float32),
            jax.ShapeDtypeStruct((T, 2), jnp.float32),
            jax.ShapeDtypeStruct((T, 2), jnp.int32),
        ],
        interpret=interpret,
    )(mm, qf, W_qe1, b_qe1, W_qe2, b_qe2, W_fg, b_fg, W_g1, W_g2)


def kernel(multimodal_feat, query_feat, W_qe1, b_qe1, W_qe2, b_qe2,
           W_fg, b_fg, W_g1, W_g2):
    logits, ew, tkw, tki = _router(
        multimodal_feat, query_feat,
        W_qe1, b_qe1.reshape(1, H),
        W_qe2, b_qe2.reshape(1, H),
        W_fg, b_fg.reshape(1, H),
        W_g1, W_g2)
    return (logits, ew, tkw, tki)
